# baseline (speedup 1.0000x reference)
import functools

import jax
import jax.numpy as jnp
from jax import lax
from jax.experimental import pallas as pl
from jax.experimental.pallas import tpu as pltpu


def _store_g(g_ref, y, b, h, w, cin):
    hw = h * w
    yb = y.astype(g_ref.dtype)
    col = lax.broadcasted_iota(jnp.int32, y.shape, 0) % w
    um = jnp.where(col == w - 1, 0, yb).reshape(b, hw, cin)
    up = jnp.where(col == 0, 0, yb).reshape(b, hw, cin)
    zp = yb.reshape(b, hw, cin)
    zeros = jnp.zeros((b, w + 1, 3 * cin), g_ref.dtype)
    g_ref[:, 0:w + 1, :] = zeros
    g_ref[:, hw + w - 1:hw + 2 * w, :] = zeros
    g_ref[:, w + 1:w + 1 + hw, 0:cin] = um
    g_ref[:, w:w + hw, cin:2 * cin] = zp
    g_ref[:, w - 1:w - 1 + hw, 2 * cin:3 * cin] = up


def _conv_from_g(g_ref, wt, bias, b, h, w, cin, rows, stride, par):
    k3 = 3 * cin
    acc = None
    for kh in range(3):
        sl = g_ref[:, pl.ds(kh * w + par, rows, stride), :]
        d = jnp.dot(sl.reshape(b * rows, k3), wt[kh * k3:(kh + 1) * k3, :],
                    preferred_element_type=jnp.float32)
        acc = d if acc is None else acc + d
    return jnp.maximum(acc + bias, 0.0)


def _tower_kernel(g1_ref, w1_ref, b1_ref, w2_ref, b2_ref, w3_ref, b3_ref,
                  o_ref, g2_ref, g3_ref, *, b, h, w):
    hw = h * w
    w1, w2, w3 = w1_ref[...], w2_ref[...], w3_ref[...]
    b1, b2, b3 = b1_ref[...], b2_ref[...], b3_ref[...]

    x27 = jnp.swapaxes(g1_ref[...], 1, 2)
    acc = jnp.dot(x27.reshape(b * hw, 27), w1,
                  preferred_element_type=jnp.float32)
    y1 = jnp.maximum(acc + b1, 0.0)
    _store_g(g2_ref, y1, b, h, w, 16)

    y2e = _conv_from_g(g2_ref, w2, b2, b, h, w, 16, hw // 2, 2, 0)
    y2o = _conv_from_g(g2_ref, w2, b2, b, h, w, 16, hw // 2, 2, 1)
    hm = jnp.maximum(y2e, y2o)
    h2, w2s = h // 2, w // 2
    r = hm.reshape(b * h2, 2, w2s, 32)
    p1 = jnp.maximum(r[:, 0], r[:, 1]).reshape(b * h2 * w2s, 32)
    _store_g(g3_ref, p1, b, h2, w2s, 32)

    hw2 = h2 * w2s
    y3e = _conv_from_g(g3_ref, w3, b3, b, h2, w2s, 32, hw2 // 2, 2, 0)
    y3o = _conv_from_g(g3_ref, w3, b3, b, h2, w2s, 32, hw2 // 2, 2, 1)
    hm2 = jnp.maximum(y3e, y3o)
    h4, w4 = h // 4, w // 4
    r2 = hm2.reshape(b * h4, 2, w4, 64)
    p2 = jnp.maximum(r2[:, 0], r2[:, 1])
    o_ref[...] = p2.reshape(b, h4 * w4, 64).astype(o_ref.dtype)


def _head_kernel(f_ref, w1_ref, b1_ref, w2_ref, b2_ref, o_ref):
    hcur = jnp.dot(f_ref[...], w1_ref[...], preferred_element_type=jnp.float32)
    hcur = jnp.maximum(hcur + b1_ref[...], 0.0)
    out = jnp.dot(hcur, w2_ref[...], preferred_element_type=jnp.float32)
    o_ref[...] = out + b2_ref[...]


@jax.jit
def _forward(x, w1, b1, w2, b2, w3, b3, wf1, bf1, wf2, bf2):
    n, _, h, w = x.shape
    hw = h * w
    h4w4 = (h // 4) * (w // 4)
    blk = 16
    cdt = jnp.bfloat16

    z1t = x.reshape(n, 3, hw).astype(cdt)
    jj = (jnp.arange(hw) % w)[None, None, :]
    um1t = jnp.where(jj == w - 1, 0, z1t)
    up1t = jnp.where(jj == 0, 0, z1t)
    pw = w + 1

    def shift(piece, d):
        zp = jnp.pad(piece, ((0, 0), (0, 0), (pw, pw)))
        return lax.slice(zp, (0, 0, pw + d), (n, 3, pw + d + hw))

    g1 = jnp.concatenate(
        [shift((um1t, z1t, up1t)[kw + 1], kh * w + kw)
         for kh in (-1, 0, 1) for kw in (-1, 0, 1)], axis=1)

    def taps(w_oihw):
        cout, cin = w_oihw.shape[0], w_oihw.shape[1]
        return jnp.transpose(w_oihw, (2, 3, 1, 0)).reshape(9 * cin, cout)

    kern = functools.partial(_tower_kernel, b=blk, h=h, w=w)
    feats = pl.pallas_call(
        kern,
        out_shape=jax.ShapeDtypeStruct((n, h4w4, 64), cdt),
        grid=(n // blk,),
        in_specs=[
            pl.BlockSpec((blk, 27, hw), lambda i: (i, 0, 0)),
            pl.BlockSpec((27, 16), lambda i: (0, 0)),
            pl.BlockSpec((1, 16), lambda i: (0, 0)),
            pl.BlockSpec((144, 32), lambda i: (0, 0)),
            pl.BlockSpec((1, 32), lambda i: (0, 0)),
            pl.BlockSpec((288, 64), lambda i: (0, 0)),
            pl.BlockSpec((1, 64), lambda i: (0, 0)),
        ],
        out_specs=pl.BlockSpec((blk, h4w4, 64), lambda i: (i, 0, 0)),
        scratch_shapes=[
            pltpu.VMEM((blk, hw + 2 * w, 48), jnp.float32),
            pltpu.VMEM((blk, hw // 4 + w, 96), jnp.float32),
        ],
        compiler_params=pltpu.CompilerParams(
            dimension_semantics=("parallel",)),
    )(g1, taps(w1).astype(cdt), b1.reshape(1, 16),
      taps(w2), b2.reshape(1, 32),
      taps(w3), b3.reshape(1, 64))

    feat = h4w4 * 64
    f2 = feats.reshape(n, feat)
    wf1f = (wf1.reshape(64, h4w4, 128).transpose(1, 0, 2)
            .reshape(feat, 128).astype(cdt))
    fblk = n // 2
    return pl.pallas_call(
        _head_kernel,
        out_shape=jax.ShapeDtypeStruct((n, 2), jnp.float32),
        grid=(2,),
        in_specs=[
            pl.BlockSpec((fblk, feat), lambda i: (i, 0)),
            pl.BlockSpec((feat, 128), lambda i: (0, 0)),
            pl.BlockSpec((1, 128), lambda i: (0, 0)),
            pl.BlockSpec((128, 2), lambda i: (0, 0)),
            pl.BlockSpec((1, 2), lambda i: (0, 0)),
        ],
        out_specs=pl.BlockSpec((fblk, 2), lambda i: (i, 0)),
        compiler_params=pltpu.CompilerParams(
            dimension_semantics=("parallel",)),
    )(f2, wf1f, bf1.reshape(1, 128), wf2, bf2.reshape(1, 2))


def kernel(x, w1, b1, w2, b2, w3, b3, wf1, bf1, wf2, bf2):
    return _forward(x, w1, b1, w2, b2, w3, b3, wf1, bf1, wf2, bf2)


# speedup vs baseline: 1.4442x; 1.4442x over previous
import functools

import jax
import jax.numpy as jnp
from jax import lax
from jax.experimental import pallas as pl
from jax.experimental.pallas import tpu as pltpu


def _store_g(g_ref, y, b, h, w, cin):
    hw = h * w
    yb = y.astype(g_ref.dtype)
    col = lax.broadcasted_iota(jnp.int32, y.shape, 0) % w
    um = jnp.where(col == w - 1, 0, yb).reshape(b, hw, cin)
    up = jnp.where(col == 0, 0, yb).reshape(b, hw, cin)
    zp = yb.reshape(b, hw, cin)
    zeros = jnp.zeros((b, w + 1, 3 * cin), g_ref.dtype)
    g_ref[:, 0:w + 1, :] = zeros
    g_ref[:, hw + w - 1:hw + 2 * w, :] = zeros
    g_ref[:, w + 1:w + 1 + hw, 0:cin] = um
    g_ref[:, w:w + hw, cin:2 * cin] = zp
    g_ref[:, w - 1:w - 1 + hw, 2 * cin:3 * cin] = up


def _conv_from_g(g_ref, wt, bias, b, h, w, cin, rows, stride, par):
    k3 = 3 * cin
    acc = None
    for kh in range(3):
        sl = g_ref[:, pl.ds(kh * w + par, rows, stride), :]
        d = jnp.dot(sl.reshape(b * rows, k3), wt[kh * k3:(kh + 1) * k3, :],
                    preferred_element_type=jnp.float32)
        acc = d if acc is None else acc + d
    return jnp.maximum(acc + bias, 0.0)


def _tower_kernel(g1_ref, w1_ref, b1_ref, w2_ref, b2_ref, w3_ref, b3_ref,
                  o_ref, g2_ref, g3_ref, *, b, h, w):
    hw = h * w
    w1, w2, w3 = w1_ref[...], w2_ref[...], w3_ref[...]
    b1, b2, b3 = b1_ref[...], b2_ref[...], b3_ref[...]

    g1 = jnp.swapaxes(g1_ref[...], 1, 2)
    acc = None
    for kh in range(3):
        sl = lax.slice(g1, (0, kh * w, 0), (b, kh * w + hw, 9))
        d = jnp.dot(sl.reshape(b * hw, 9), w1[kh * 9:(kh + 1) * 9, :],
                    preferred_element_type=jnp.float32)
        acc = d if acc is None else acc + d
    y1 = jnp.maximum(acc + b1, 0.0)
    _store_g(g2_ref, y1, b, h, w, 16)

    y2e = _conv_from_g(g2_ref, w2, b2, b, h, w, 16, hw // 2, 2, 0)
    y2o = _conv_from_g(g2_ref, w2, b2, b, h, w, 16, hw // 2, 2, 1)
    hm = jnp.maximum(y2e, y2o)
    h2, w2s = h // 2, w // 2
    r = hm.reshape(b * h2, 2, w2s, 32)
    p1 = jnp.maximum(r[:, 0], r[:, 1]).reshape(b * h2 * w2s, 32)
    _store_g(g3_ref, p1, b, h2, w2s, 32)

    hw2 = h2 * w2s
    y3e = _conv_from_g(g3_ref, w3, b3, b, h2, w2s, 32, hw2 // 2, 2, 0)
    y3o = _conv_from_g(g3_ref, w3, b3, b, h2, w2s, 32, hw2 // 2, 2, 1)
    hm2 = jnp.maximum(y3e, y3o)
    h4, w4 = h // 4, w // 4
    r2 = hm2.reshape(b * h4, 2, w4, 64)
    p2 = jnp.maximum(r2[:, 0], r2[:, 1])
    o_ref[...] = p2.reshape(b, h4 * w4, 64).astype(o_ref.dtype)


def _head_kernel(f_ref, w1_ref, b1_ref, w2_ref, b2_ref, o_ref):
    hcur = jnp.dot(f_ref[...], w1_ref[...], preferred_element_type=jnp.float32)
    hcur = jnp.maximum(hcur + b1_ref[...], 0.0)
    out = jnp.dot(hcur, w2_ref[...], preferred_element_type=jnp.float32)
    o_ref[...] = out + b2_ref[...]


@jax.jit
def _forward(x, w1, b1, w2, b2, w3, b3, wf1, bf1, wf2, bf2):
    n, _, h, w = x.shape
    hw = h * w
    h4w4 = (h // 4) * (w // 4)
    blk = 16
    cdt = jnp.bfloat16

    z1t = x.reshape(n, 3, hw).astype(cdt)
    jj = (jnp.arange(hw) % w)[None, None, :]
    um1t = jnp.where(jj == w - 1, 0, z1t)
    up1t = jnp.where(jj == 0, 0, z1t)
    rg = hw + 2 * w

    def place(piece, t0):
        return jnp.pad(piece, ((0, 0), (0, 0), (t0, rg - hw - t0)))

    g1 = jnp.concatenate(
        [place(um1t, w + 1), place(z1t, w), place(up1t, w - 1)], axis=1)

    def taps(w_oihw):
        cout, cin = w_oihw.shape[0], w_oihw.shape[1]
        return jnp.transpose(w_oihw, (2, 3, 1, 0)).reshape(9 * cin, cout)

    kern = functools.partial(_tower_kernel, b=blk, h=h, w=w)
    feats = pl.pallas_call(
        kern,
        out_shape=jax.ShapeDtypeStruct((n, h4w4, 64), cdt),
        grid=(n // blk,),
        in_specs=[
            pl.BlockSpec((blk, 9, rg), lambda i: (i, 0, 0)),
            pl.BlockSpec((27, 16), lambda i: (0, 0)),
            pl.BlockSpec((1, 16), lambda i: (0, 0)),
            pl.BlockSpec((144, 32), lambda i: (0, 0)),
            pl.BlockSpec((1, 32), lambda i: (0, 0)),
            pl.BlockSpec((288, 64), lambda i: (0, 0)),
            pl.BlockSpec((1, 64), lambda i: (0, 0)),
        ],
        out_specs=pl.BlockSpec((blk, h4w4, 64), lambda i: (i, 0, 0)),
        scratch_shapes=[
            pltpu.VMEM((blk, hw + 2 * w, 48), jnp.float32),
            pltpu.VMEM((blk, hw // 4 + w, 96), jnp.float32),
        ],
        compiler_params=pltpu.CompilerParams(
            dimension_semantics=("parallel",)),
    )(g1, taps(w1).astype(cdt), b1.reshape(1, 16),
      taps(w2), b2.reshape(1, 32),
      taps(w3), b3.reshape(1, 64))

    feat = h4w4 * 64
    f2 = feats.reshape(n, feat)
    wf1f = (wf1.reshape(64, h4w4, 128).transpose(1, 0, 2)
            .reshape(feat, 128).astype(cdt))
    fblk = n // 2
    return pl.pallas_call(
        _head_kernel,
        out_shape=jax.ShapeDtypeStruct((n, 2), jnp.float32),
        grid=(2,),
        in_specs=[
            pl.BlockSpec((fblk, feat), lambda i: (i, 0)),
            pl.BlockSpec((feat, 128), lambda i: (0, 0)),
            pl.BlockSpec((1, 128), lambda i: (0, 0)),
            pl.BlockSpec((128, 2), lambda i: (0, 0)),
            pl.BlockSpec((1, 2), lambda i: (0, 0)),
        ],
        out_specs=pl.BlockSpec((fblk, 2), lambda i: (i, 0)),
        compiler_params=pltpu.CompilerParams(
            dimension_semantics=("parallel",)),
    )(f2, wf1f, bf1.reshape(1, 128), wf2, bf2.reshape(1, 2))


def kernel(x, w1, b1, w2, b2, w3, b3, wf1, bf1, wf2, bf2):
    return _forward(x, w1, b1, w2, b2, w3, b3, wf1, bf1, wf2, bf2)


# speedup vs baseline: 1.4534x; 1.0064x over previous
import functools

import jax
import jax.numpy as jnp
from jax import lax
from jax.experimental import pallas as pl
from jax.experimental.pallas import tpu as pltpu


def _store_g(g_ref, y, b, h, w, cin):
    hw = h * w
    yb = y.astype(g_ref.dtype)
    col = lax.broadcasted_iota(jnp.int32, y.shape, 0) % w
    um = jnp.where(col == w - 1, 0, yb).reshape(b, hw, cin)
    up = jnp.where(col == 0, 0, yb).reshape(b, hw, cin)
    zp = yb.reshape(b, hw, cin)
    zeros = jnp.zeros((b, w + 1, 3 * cin), g_ref.dtype)
    g_ref[:, 0:w + 1, :] = zeros
    g_ref[:, hw + w - 1:hw + 2 * w, :] = zeros
    g_ref[:, w + 1:w + 1 + hw, 0:cin] = um
    g_ref[:, w:w + hw, cin:2 * cin] = zp
    g_ref[:, w - 1:w - 1 + hw, 2 * cin:3 * cin] = up


def _conv_from_g(g_ref, wt, bias, b, h, w, cin, rows, stride, par):
    k3 = 3 * cin
    acc = None
    for kh in range(3):
        sl = g_ref[:, pl.ds(kh * w + par, rows, stride), :]
        d = jnp.dot(sl.reshape(b * rows, k3), wt[kh * k3:(kh + 1) * k3, :],
                    preferred_element_type=jnp.float32)
        acc = d if acc is None else acc + d
    return jnp.maximum(acc + bias, 0.0)


def _tower_half(g1, w1, b1, w2, b2, w3, b3, g2_ref, g3_ref, b, h, w):
    hw = h * w
    acc = None
    for kh in range(3):
        sl = lax.slice(g1, (0, kh * w, 0), (b, kh * w + hw, 9))
        d = jnp.dot(sl.reshape(b * hw, 9), w1[kh * 9:(kh + 1) * 9, :],
                    preferred_element_type=jnp.float32)
        acc = d if acc is None else acc + d
    y1 = jnp.maximum(acc + b1, 0.0)
    _store_g(g2_ref, y1, b, h, w, 16)

    y2e = _conv_from_g(g2_ref, w2, b2, b, h, w, 16, hw // 2, 2, 0)
    y2o = _conv_from_g(g2_ref, w2, b2, b, h, w, 16, hw // 2, 2, 1)
    hm = jnp.maximum(y2e, y2o)
    h2, w2s = h // 2, w // 2
    r = hm.reshape(b * h2, 2, w2s, 32)
    p1 = jnp.maximum(r[:, 0], r[:, 1]).reshape(b * h2 * w2s, 32)
    _store_g(g3_ref, p1, b, h2, w2s, 32)

    hw2 = h2 * w2s
    y3e = _conv_from_g(g3_ref, w3, b3, b, h2, w2s, 32, hw2 // 2, 2, 0)
    y3o = _conv_from_g(g3_ref, w3, b3, b, h2, w2s, 32, hw2 // 2, 2, 1)
    hm2 = jnp.maximum(y3e, y3o)
    h4, w4 = h // 4, w // 4
    r2 = hm2.reshape(b * h4, 2, w4, 64)
    p2 = jnp.maximum(r2[:, 0], r2[:, 1])
    return p2.reshape(b, h4 * w4, 64)


def _tower_kernel(g1_ref, w1_ref, b1_ref, w2_ref, b2_ref, w3_ref, b3_ref,
                  o_ref, g2_ref, g3_ref, *, b, h, w):
    w1, w2, w3 = w1_ref[...], w2_ref[...], w3_ref[...]
    b1, b2, b3 = b1_ref[...], b2_ref[...], b3_ref[...]
    g1 = jnp.swapaxes(g1_ref[...], 1, 2)
    p2 = _tower_half(g1, w1, b1, w2, b2, w3, b3, g2_ref, g3_ref, b, h, w)
    o_ref[...] = p2.astype(o_ref.dtype)


def _head_kernel(f_ref, w1_ref, b1_ref, w2_ref, b2_ref, o_ref):
    hcur = jnp.dot(f_ref[...], w1_ref[...], preferred_element_type=jnp.float32)
    hcur = jnp.maximum(hcur + b1_ref[...], 0.0)
    out = jnp.dot(hcur, w2_ref[...], preferred_element_type=jnp.float32)
    o_ref[...] = out + b2_ref[...]


@jax.jit
def _forward(x, w1, b1, w2, b2, w3, b3, wf1, bf1, wf2, bf2):
    n, _, h, w = x.shape
    hw = h * w
    h4w4 = (h // 4) * (w // 4)
    blk = 32
    cdt = jnp.bfloat16

    z1t = x.reshape(n, 3, hw).astype(cdt)
    jj = (jnp.arange(hw) % w)[None, None, :]
    um1t = jnp.where(jj == w - 1, 0, z1t)
    up1t = jnp.where(jj == 0, 0, z1t)
    rg = hw + 2 * w

    def place(piece, t0):
        return jnp.pad(piece, ((0, 0), (0, 0), (t0, rg - hw - t0)))

    g1 = jnp.concatenate(
        [place(um1t, w + 1), place(z1t, w), place(up1t, w - 1)], axis=1)

    def taps(w_oihw):
        cout, cin = w_oihw.shape[0], w_oihw.shape[1]
        return jnp.transpose(w_oihw, (2, 3, 1, 0)).reshape(9 * cin, cout)

    kern = functools.partial(_tower_kernel, b=blk, h=h, w=w)
    feats = pl.pallas_call(
        kern,
        out_shape=jax.ShapeDtypeStruct((n, h4w4, 64), cdt),
        grid=(n // blk,),
        in_specs=[
            pl.BlockSpec((blk, 9, rg), lambda i: (i, 0, 0)),
            pl.BlockSpec((27, 16), lambda i: (0, 0)),
            pl.BlockSpec((1, 16), lambda i: (0, 0)),
            pl.BlockSpec((144, 32), lambda i: (0, 0)),
            pl.BlockSpec((1, 32), lambda i: (0, 0)),
            pl.BlockSpec((288, 64), lambda i: (0, 0)),
            pl.BlockSpec((1, 64), lambda i: (0, 0)),
        ],
        out_specs=pl.BlockSpec((blk, h4w4, 64), lambda i: (i, 0, 0)),
        scratch_shapes=[
            pltpu.VMEM((blk, hw + 2 * w, 48), jnp.float32),
            pltpu.VMEM((blk, hw // 4 + w, 96), jnp.float32),
        ],
        compiler_params=pltpu.CompilerParams(
            dimension_semantics=("parallel",)),
    )(g1, taps(w1).astype(cdt), b1.reshape(1, 16),
      taps(w2), b2.reshape(1, 32),
      taps(w3), b3.reshape(1, 64))

    feat = h4w4 * 64
    f2 = feats.reshape(n, feat)
    wf1f = (wf1.reshape(64, h4w4, 128).transpose(1, 0, 2)
            .reshape(feat, 128).astype(cdt))
    fblk = n // 2
    return pl.pallas_call(
        _head_kernel,
        out_shape=jax.ShapeDtypeStruct((n, 2), jnp.float32),
        grid=(2,),
        in_specs=[
            pl.BlockSpec((fblk, feat), lambda i: (i, 0)),
            pl.BlockSpec((feat, 128), lambda i: (0, 0)),
            pl.BlockSpec((1, 128), lambda i: (0, 0)),
            pl.BlockSpec((128, 2), lambda i: (0, 0)),
            pl.BlockSpec((1, 2), lambda i: (0, 0)),
        ],
        out_specs=pl.BlockSpec((fblk, 2), lambda i: (i, 0)),
        compiler_params=pltpu.CompilerParams(
            dimension_semantics=("parallel",)),
    )(f2, wf1f, bf1.reshape(1, 128), wf2, bf2.reshape(1, 2))


def kernel(x, w1, b1, w2, b2, w3, b3, wf1, bf1, wf2, bf2):
    return _forward(x, w1, b1, w2, b2, w3, b3, wf1, bf1, wf2, bf2)


# speedup vs baseline: 2.2838x; 1.5714x over previous
import functools

import jax
import jax.numpy as jnp
from jax import lax
from jax.experimental import pallas as pl
from jax.experimental.pallas import tpu as pltpu


def _store_g(g_ref, y, b, h, w, cin):
    hw = h * w
    yb = y.astype(g_ref.dtype)
    col = lax.broadcasted_iota(jnp.int32, y.shape, 0) % w
    um = jnp.where(col == w - 1, 0, yb).reshape(b, hw, cin)
    up = jnp.where(col == 0, 0, yb).reshape(b, hw, cin)
    zp = yb.reshape(b, hw, cin)
    zeros = jnp.zeros((b, w + 1, 3 * cin), g_ref.dtype)
    g_ref[:, 0:w + 1, :] = zeros
    g_ref[:, hw + w - 1:hw + 2 * w, :] = zeros
    g_ref[:, w + 1:w + 1 + hw, 0:cin] = um
    g_ref[:, w:w + hw, cin:2 * cin] = zp
    g_ref[:, w - 1:w - 1 + hw, 2 * cin:3 * cin] = up


def _sum3_relu(p3, bias, b, rows, sh):
    s = None
    for kh in range(3):
        t = lax.slice(p3, (0, sh * kh, kh * 128),
                      (b, sh * kh + rows, kh * 128 + 128))
        s = t if s is None else s + t
    return jnp.maximum(s + bias, 0.0)


def _conv_from_g(g_ref, wt_big, bias, b, k3, rows, sh, par):
    vext = rows + 2 * sh
    sl = g_ref[:, pl.ds(par, vext, 2), :]
    p = jnp.dot(sl.reshape(b * vext, k3), wt_big,
                preferred_element_type=jnp.float32)
    return _sum3_relu(p.reshape(b, vext, 384), bias, b, rows, sh)


def _tower_half(g1, w1, b1, w2, b2, w3, b3, g2_ref, g3_ref, b, h, w):
    hw = h * w
    rg = hw + 2 * w
    p1w = jnp.dot(g1.reshape(b * rg, 9), w1,
                  preferred_element_type=jnp.float32)
    y1 = _sum3_relu(p1w.reshape(b, rg, 384), b1, b, hw, w)
    _store_g(g2_ref, lax.slice(y1, (0, 0, 0), (b, hw, 16))
             .reshape(b * hw, 16), b, h, w, 16)

    y2e = _conv_from_g(g2_ref, w2, b2, b, 48, hw // 2, w // 2, 0)
    y2o = _conv_from_g(g2_ref, w2, b2, b, 48, hw // 2, w // 2, 1)
    hm = jnp.maximum(y2e, y2o)
    h2, w2s = h // 2, w // 2
    hm32 = lax.slice(hm, (0, 0, 0), (b, hw // 2, 32))
    r = hm32.reshape(b * h2, 2, w2s, 32)
    p1 = jnp.maximum(r[:, 0], r[:, 1]).reshape(b * h2 * w2s, 32)
    _store_g(g3_ref, p1, b, h2, w2s, 32)

    hw2 = h2 * w2s
    y3e = _conv_from_g(g3_ref, w3, b3, b, 96, hw2 // 2, w2s // 2, 0)
    y3o = _conv_from_g(g3_ref, w3, b3, b, 96, hw2 // 2, w2s // 2, 1)
    hm2 = jnp.maximum(y3e, y3o)
    h4, w4 = h // 4, w // 4
    hm64 = lax.slice(hm2, (0, 0, 0), (b, hw2 // 2, 64))
    r2 = hm2.reshape(1, 1, 1, 1) if False else hm64.reshape(b * h4, 2, w4, 64)
    p2 = jnp.maximum(r2[:, 0], r2[:, 1])
    return p2.reshape(b, h4 * w4, 64)


def _tower_kernel(g1_ref, w1_ref, b1_ref, w2_ref, b2_ref, w3_ref, b3_ref,
                  o_ref, g2_ref, g3_ref, *, b, h, w):
    w1, w2, w3 = w1_ref[...], w2_ref[...], w3_ref[...]
    b1, b2, b3 = b1_ref[...], b2_ref[...], b3_ref[...]
    g1 = jnp.swapaxes(g1_ref[...], 1, 2)
    p2 = _tower_half(g1, w1, b1, w2, b2, w3, b3, g2_ref, g3_ref, b, h, w)
    o_ref[...] = p2.astype(o_ref.dtype)


def _head_kernel(f_ref, w1_ref, b1_ref, w2_ref, b2_ref, o_ref):
    hcur = jnp.dot(f_ref[...], w1_ref[...], preferred_element_type=jnp.float32)
    hcur = jnp.maximum(hcur + b1_ref[...], 0.0)
    out = jnp.dot(hcur, w2_ref[...], preferred_element_type=jnp.float32)
    o_ref[...] = out + b2_ref[...]


@jax.jit
def _forward(x, w1, b1, w2, b2, w3, b3, wf1, bf1, wf2, bf2):
    n, _, h, w = x.shape
    hw = h * w
    h4w4 = (h // 4) * (w // 4)
    blk = 16
    cdt = jnp.bfloat16

    z1t = x.reshape(n, 3, hw).astype(cdt)
    jj = (jnp.arange(hw) % w)[None, None, :]
    um1t = jnp.where(jj == w - 1, 0, z1t)
    up1t = jnp.where(jj == 0, 0, z1t)
    rg = hw + 2 * w

    def place(piece, t0):
        return jnp.pad(piece, ((0, 0), (0, 0), (t0, rg - hw - t0)))

    g1 = jnp.concatenate(
        [place(um1t, w + 1), place(z1t, w), place(up1t, w - 1)], axis=1)

    def taps(w_oihw):
        cout, cin = w_oihw.shape[0], w_oihw.shape[1]
        t = jnp.transpose(w_oihw, (2, 3, 1, 0)).reshape(9 * cin, cout)
        k3 = 3 * cin
        return jnp.concatenate(
            [jnp.pad(t[kh * k3:(kh + 1) * k3], ((0, 0), (0, 128 - cout)))
             for kh in range(3)], axis=1)

    def bpad(bv):
        return jnp.pad(bv.reshape(1, -1), ((0, 0), (0, 128 - bv.shape[0])))

    kern = functools.partial(_tower_kernel, b=blk, h=h, w=w)
    feats = pl.pallas_call(
        kern,
        out_shape=jax.ShapeDtypeStruct((n, h4w4, 64), cdt),
        grid=(n // blk,),
        in_specs=[
            pl.BlockSpec((blk, 9, rg), lambda i: (i, 0, 0)),
            pl.BlockSpec((9, 384), lambda i: (0, 0)),
            pl.BlockSpec((1, 128), lambda i: (0, 0)),
            pl.BlockSpec((48, 384), lambda i: (0, 0)),
            pl.BlockSpec((1, 128), lambda i: (0, 0)),
            pl.BlockSpec((96, 384), lambda i: (0, 0)),
            pl.BlockSpec((1, 128), lambda i: (0, 0)),
        ],
        out_specs=pl.BlockSpec((blk, h4w4, 64), lambda i: (i, 0, 0)),
        scratch_shapes=[
            pltpu.VMEM((blk, hw + 2 * w, 48), jnp.float32),
            pltpu.VMEM((blk, hw // 4 + w, 96), jnp.float32),
        ],
        compiler_params=pltpu.CompilerParams(
            dimension_semantics=("parallel",)),
    )(g1, taps(w1).astype(cdt), bpad(b1),
      taps(w2), bpad(b2),
      taps(w3), bpad(b3))

    feat = h4w4 * 64
    f2 = feats.reshape(n, feat)
    wf1f = (wf1.reshape(64, h4w4, 128).transpose(1, 0, 2)
            .reshape(feat, 128).astype(cdt))
    fblk = n // 2
    return pl.pallas_call(
        _head_kernel,
        out_shape=jax.ShapeDtypeStruct((n, 2), jnp.float32),
        grid=(2,),
        in_specs=[
            pl.BlockSpec((fblk, feat), lambda i: (i, 0)),
            pl.BlockSpec((feat, 128), lambda i: (0, 0)),
            pl.BlockSpec((1, 128), lambda i: (0, 0)),
            pl.BlockSpec((128, 2), lambda i: (0, 0)),
            pl.BlockSpec((1, 2), lambda i: (0, 0)),
        ],
        out_specs=pl.BlockSpec((fblk, 2), lambda i: (i, 0)),
        compiler_params=pltpu.CompilerParams(
            dimension_semantics=("parallel",)),
    )(f2, wf1f, bf1.reshape(1, 128), wf2, bf2.reshape(1, 2))


def kernel(x, w1, b1, w2, b2, w3, b3, wf1, bf1, wf2, bf2):
    return _forward(x, w1, b1, w2, b2, w3, b3, wf1, bf1, wf2, bf2)
